# baseline (speedup 1.0000x reference)
import functools

import jax
import jax.numpy as jnp
from jax.experimental import pallas as pl
from jax.experimental.pallas import tpu as pltpu


def _bn_lrelu(x, slope):
    m = jnp.mean(x, axis=0)
    v = jnp.var(x, axis=0)
    return jax.nn.leaky_relu((x - m) / jnp.sqrt(v + 1e-5), slope)


def _mlp(x, ws, slope):
    for w in ws:
        x = _bn_lrelu(x @ w.T, slope)
    return x


def _seg_max(m, dst, n):
    agg = jax.ops.segment_max(m, dst, num_segments=n)
    return jnp.where(jnp.isfinite(agg), agg, 0.0)


def _static_conv(x, edge_index, edge_attr, p, slope=0.1):
    src, dst = edge_index[0], edge_index[1]
    x_i = x[dst]
    x_j = x[src]
    m = jnp.concatenate([x_i, x_j - x_i, edge_attr], axis=1)
    m = _mlp(m, p['msg'], slope)
    return _mlp(x, p['lin'], slope) + _seg_max(m, dst, x.shape[0])


def _knn_edges(x, k):
    xs = jax.lax.stop_gradient(x)
    sq = jnp.sum(xs * xs, axis=1)
    d = sq[:, None] + sq[None, :] - 2.0 * (xs @ xs.T)
    n = xs.shape[0]
    d = d.at[jnp.arange(n), jnp.arange(n)].set(jnp.inf)
    _, nn_idx = jax.lax.top_k(-d, k)
    dst = jnp.repeat(jnp.arange(n), k)
    src = nn_idx.reshape(-1)
    src = jnp.concatenate([src, jnp.arange(n)])
    dst = jnp.concatenate([dst, jnp.arange(n)])
    return src, dst


def _dynamic_conv(x, k, p, slope=0.2):
    src, dst = _knn_edges(x, 8)
    x_i = x[dst]
    x_j = x[src]
    m = jnp.concatenate([x_i, x_j - x_i], axis=1)
    m = _mlp(m, p['msg'], slope)
    return _mlp(x, p['lin'], slope) + _seg_max(m, dst, x.shape[0])


def _mm_kernel(x_ref, w_ref, o_ref):
    o_ref[...] = jax.lax.dot_general(
        x_ref[...], w_ref[...], (((1,), (1,)), ((), ())),
        preferred_element_type=jnp.float32)


def _pallas_mm_t(x, w):
    R, Cin = x.shape
    Cout = w.shape[0]
    BR = 512
    grid = (pl.cdiv(R, BR),)
    return pl.pallas_call(
        _mm_kernel,
        grid=grid,
        in_specs=[pl.BlockSpec((BR, Cin), lambda i: (i, 0)),
                  pl.BlockSpec((Cout, Cin), lambda i: (0, 0))],
        out_specs=pl.BlockSpec((BR, Cout), lambda i: (i, 0)),
        out_shape=jax.ShapeDtypeStruct((R, Cout), jnp.float32),
    )(x, w)


def kernel(node_embedding, edge_embedding, edge_index, k, params):
    s1 = _static_conv(node_embedding, edge_index, edge_embedding, params['sg1'], 0.1)
    s2 = _static_conv(s1, edge_index, edge_embedding, params['sg2'], 0.1)
    s3 = _static_conv(s2, edge_index, edge_embedding, params['sg3'], 0.1)
    d1 = _dynamic_conv(s1, k, params['dg1'], 0.2)
    d2 = _dynamic_conv(d1, k, params['dg2'], 0.2)
    cat1 = jnp.concatenate([s1, d1, d2, s2, s3], axis=1)
    f1 = _bn_lrelu(cat1 @ params['fuse1'][0].T, 0.2)
    cat2 = jnp.concatenate([f1, cat1], axis=1)
    h = _bn_lrelu(cat2 @ params['fuse2'][0].T, 0.2)
    h = _bn_lrelu(h @ params['fuse2'][1].T, 0.2)
    return _pallas_mm_t(h, params['fuse2'][2])


# speedup vs baseline: 1.9700x; 1.9700x over previous
import functools

import jax
import jax.numpy as jnp
from jax.experimental import pallas as pl
from jax.experimental.pallas import tpu as pltpu

_NEG = -3.0e38


def _block_stats(y, base, nrows, s_ref, q_ref):
    valid = jax.lax.broadcasted_iota(jnp.int32, y.shape, 0) + base < nrows
    ym = jnp.where(valid, y, 0.0)
    s_ref[...] = jnp.sum(ym, axis=0).reshape(s_ref.shape)
    q_ref[...] = jnp.sum(ym * ym, axis=0).reshape(q_ref.shape)


def _mm_stats_kernel(nrows, x_ref, w_ref, o_ref, s_ref, q_ref):
    i = pl.program_id(0)
    x = x_ref[...]
    y = jax.lax.dot_general(x, w_ref[...], (((1,), (1,)), ((), ())),
                            preferred_element_type=jnp.float32)
    o_ref[...] = y
    _block_stats(y, i * x.shape[0], nrows, s_ref, q_ref)


def _mm_affine_stats_kernel(nrows, slope, x_ref, sc_ref, bi_ref, w_ref,
                            o_ref, s_ref, q_ref):
    i = pl.program_id(0)
    x = (x_ref[...] - sc_ref[...]) / jnp.sqrt(bi_ref[...] + 1e-5)
    x = jnp.maximum(x, slope * x)
    y = jax.lax.dot_general(x, w_ref[...], (((1,), (1,)), ((), ())),
                            preferred_element_type=jnp.float32)
    o_ref[...] = y
    _block_stats(y, i * x.shape[0], nrows, s_ref, q_ref)


def _finalize_stats(s, q, nrows, br):
    s = s.reshape(-1, s.shape[-1])
    q = q.reshape(-1, q.shape[-1])
    m = jnp.sum(s, axis=0) / nrows
    v = jnp.sum(q, axis=0) / nrows - m * m
    return m, v


def _mm(x, w, affine=None, slope=0.1, stats=True, br=512):
    R, Cin = x.shape
    Cout = w.shape[0]
    grid = pl.cdiv(R, br)
    outs = [jax.ShapeDtypeStruct((grid * br, Cout), jnp.float32),
            jax.ShapeDtypeStruct((grid, 1, Cout), jnp.float32),
            jax.ShapeDtypeStruct((grid, 1, Cout), jnp.float32)]
    o_specs = [pl.BlockSpec((br, Cout), lambda i: (i, 0)),
               pl.BlockSpec((1, 1, Cout), lambda i: (i, 0, 0)),
               pl.BlockSpec((1, 1, Cout), lambda i: (i, 0, 0))]
    if affine is None:
        kfn = functools.partial(_mm_stats_kernel, R)
        in_specs = [pl.BlockSpec((br, Cin), lambda i: (i, 0)),
                    pl.BlockSpec((Cout, Cin), lambda i: (0, 0))]
        args = (x, w)
    else:
        s, b = affine
        kfn = functools.partial(_mm_affine_stats_kernel, R, slope)
        in_specs = [pl.BlockSpec((br, Cin), lambda i: (i, 0)),
                    pl.BlockSpec((1, Cin), lambda i: (0, 0)),
                    pl.BlockSpec((1, Cin), lambda i: (0, 0)),
                    pl.BlockSpec((Cout, Cin), lambda i: (0, 0))]
        args = (x, s.reshape(1, Cin), b.reshape(1, Cin), w)
    y, ps, pq = pl.pallas_call(
        kfn, grid=(grid,), in_specs=in_specs, out_specs=o_specs,
        out_shape=outs)(*args)
    y = y[:R]
    if not stats:
        return y, None, None
    scale, bias = _finalize_stats(ps, pq, R, br)
    return y, scale, bias


def _apply_kernel(slope, y_ref, s_ref, b_ref, o_ref):
    y = (y_ref[...] - s_ref[...]) / jnp.sqrt(b_ref[...] + 1e-5)
    o_ref[...] = jnp.maximum(y, slope * y)


def _apply(y, scale, bias, slope, br=1024):
    R, C = y.shape
    grid = pl.cdiv(R, br)
    out = pl.pallas_call(
        functools.partial(_apply_kernel, slope),
        grid=(grid,),
        in_specs=[pl.BlockSpec((br, C), lambda i: (i, 0)),
                  pl.BlockSpec((1, C), lambda i: (0, 0)),
                  pl.BlockSpec((1, C), lambda i: (0, 0))],
        out_specs=pl.BlockSpec((br, C), lambda i: (i, 0)),
        out_shape=jax.ShapeDtypeStruct((grid * br, C), jnp.float32),
    )(y, scale.reshape(1, C), bias.reshape(1, C))
    return out[:R]


def _stats_kernel(nrows, x_ref, s_ref, q_ref):
    i = pl.program_id(0)
    x = x_ref[...]
    _block_stats(x, i * x.shape[0], nrows, s_ref, q_ref)


def _stats(x, br=1024):
    R, C = x.shape
    grid = pl.cdiv(R, br)
    ps, pq = pl.pallas_call(
        functools.partial(_stats_kernel, R),
        grid=(grid,),
        in_specs=[pl.BlockSpec((br, C), lambda i: (i, 0))],
        out_specs=[pl.BlockSpec((1, 1, C), lambda i: (i, 0, 0)),
                   pl.BlockSpec((1, 1, C), lambda i: (i, 0, 0))],
        out_shape=[jax.ShapeDtypeStruct((grid, 1, C), jnp.float32),
                   jax.ShapeDtypeStruct((grid, 1, C), jnp.float32)],
    )(x)
    return _finalize_stats(ps, pq, R, br)


def _knn_kernel(n, k, xb_ref, xt_ref, sq_ref, idx_ref):
    i = pl.program_id(0)
    xb = xb_ref[...]
    br = xb.shape[0]
    sqb = jnp.sum(xb * xb, axis=1, keepdims=True)
    d = sqb + sq_ref[...] - 2.0 * jax.lax.dot_general(
        xb, xt_ref[...], (((1,), (1,)), ((), ())),
        preferred_element_type=jnp.float32)
    col = jax.lax.broadcasted_iota(jnp.int32, d.shape, 1)
    row = i * br + jax.lax.broadcasted_iota(jnp.int32, d.shape, 0)
    d = jnp.where(col == row, jnp.inf, d)
    cols = []
    for j in range(k):
        mn = jnp.min(d, axis=1, keepdims=True)
        am = jnp.min(jnp.where(d == mn, col, n), axis=1, keepdims=True)
        cols.append(am)
        d = jnp.where(col == am, jnp.inf, d)
    idx_ref[...] = jnp.concatenate(cols, axis=1)


def _knn(x, k=8, br=256):
    n, c = x.shape
    sq = jnp.sum(x * x, axis=1)
    grid = pl.cdiv(n, br)
    idx = pl.pallas_call(
        functools.partial(_knn_kernel, n, k),
        grid=(grid,),
        in_specs=[pl.BlockSpec((br, c), lambda i: (i, 0)),
                  pl.BlockSpec((n, c), lambda i: (0, 0)),
                  pl.BlockSpec((1, n), lambda i: (0, 0))],
        out_specs=pl.BlockSpec((br, k), lambda i: (i, 0)),
        out_shape=jax.ShapeDtypeStruct((grid * br, k), jnp.int32),
    )(x, x, sq.reshape(1, n))
    return idx[:n]


def _dyn_reduce_kernel(slope, y_ref, s_ref, b_ref, l_ref, ls_ref, lb_ref,
                       o_ref):
    y = (y_ref[...] - s_ref[...]) / jnp.sqrt(b_ref[...] + 1e-5)
    y = jnp.maximum(y, slope * y)
    bn = y.shape[0] // 9
    y = jnp.max(y.reshape(bn, 9, y.shape[1]), axis=1)
    l = (l_ref[...] - ls_ref[...]) / jnp.sqrt(lb_ref[...] + 1e-5)
    l = jnp.maximum(l, slope * l)
    o_ref[...] = l + y


def _dyn_reduce(y, ys, yb, lin, ls, lb, slope, bn=256):
    R9, C = y.shape
    N = R9 // 9
    grid = pl.cdiv(N, bn)
    pad = grid * bn
    if pad != N:
        y = jnp.pad(y, ((0, (pad - N) * 9), (0, 0)))
        lin = jnp.pad(lin, ((0, pad - N), (0, 0)))
    out = pl.pallas_call(
        functools.partial(_dyn_reduce_kernel, slope),
        grid=(grid,),
        in_specs=[pl.BlockSpec((bn * 9, C), lambda i: (i, 0)),
                  pl.BlockSpec((1, C), lambda i: (0, 0)),
                  pl.BlockSpec((1, C), lambda i: (0, 0)),
                  pl.BlockSpec((bn, C), lambda i: (i, 0)),
                  pl.BlockSpec((1, C), lambda i: (0, 0)),
                  pl.BlockSpec((1, C), lambda i: (0, 0))],
        out_specs=pl.BlockSpec((bn, C), lambda i: (i, 0)),
        out_shape=jax.ShapeDtypeStruct((pad, C), jnp.float32),
    )(y, ys.reshape(1, C), yb.reshape(1, C), lin, ls.reshape(1, C),
      lb.reshape(1, C))
    return out[:N]


def _bn_lrelu_x(x, slope):
    m = jnp.mean(x, axis=0)
    v = jnp.var(x, axis=0)
    return jax.nn.leaky_relu((x - m) / jnp.sqrt(v + 1e-5), slope)


def _mlp_x(x, ws, slope):
    for w in ws:
        x = _bn_lrelu_x(x @ w.T, slope)
    return x


def _seg_max_x(m, dst, n):
    agg = jax.ops.segment_max(m, dst, num_segments=n)
    return jnp.where(jnp.isfinite(agg), agg, 0.0)


def _static_conv_x(x, src, dst, edge_attr, p, slope=0.1):
    x_i = x[dst]
    x_j = x[src]
    m = jnp.concatenate([x_i, x_j - x_i, edge_attr], axis=1)
    m = _mlp_x(m, p['msg'], slope)
    return _mlp_x(x, p['lin'], slope) + _seg_max_x(m, dst, x.shape[0])


def _dynamic_conv_x(x, nn_idx, p, slope=0.2):
    n = x.shape[0]
    ar = jnp.arange(n, dtype=nn_idx.dtype)
    src = jnp.concatenate([nn_idx.reshape(-1), ar])
    dst = jnp.concatenate([jnp.repeat(ar, nn_idx.shape[1]), ar])
    x_i = x[dst]
    x_j = x[src]
    m = jnp.concatenate([x_i, x_j - x_i], axis=1)
    m = _mlp_x(m, p['msg'], slope)
    return _mlp_x(x, p['lin'], slope) + _seg_max_x(m, dst, n)


def _split_w1(w1, c):
    wa, wb, wc = w1[:, :c], w1[:, c:2 * c], w1[:, 2 * c:]
    return wa - wb, wb, wc


def _static_conv(x, src, dst, edge_attr, p, slope=0.1):
    n, c = x.shape
    x_i = x[dst]
    x_j = x[src]
    cat = jnp.concatenate([x_i, x_j - x_i, edge_attr], axis=1)
    m1, s1c, b1c = _mm(cat, p['msg'][0])
    y2, s2c, b2c = _mm(m1, p['msg'][1], affine=(s1c, b1c), slope=slope)
    y3, s3c, b3c = _mm(y2, p['msg'][2], affine=(s2c, b2c), slope=slope)
    m3 = _apply(y3, s3c, b3c, slope)
    seg = jax.ops.segment_max(m3, dst, num_segments=n)
    seg = jnp.where(jnp.isfinite(seg), seg, 0.0)
    l1, ls1, lb1 = _mm(x, p['lin'][0])
    l2, ls2, lb2 = _mm(l1, p['lin'][1], affine=(ls1, lb1), slope=slope)
    return _apply(l2, ls2, lb2, slope) + seg


def _dynamic_conv(x, nn_idx, p, slope=0.2):
    n, c = x.shape
    idx9 = jnp.concatenate([nn_idx, jnp.arange(n, dtype=jnp.int32)[:, None]],
                           axis=1)
    x9 = x[idx9]
    xi = jnp.broadcast_to(x[:, None, :], x9.shape).reshape(n * 9, c)
    xd = (x9 - x[:, None, :]).reshape(n * 9, c)
    cat = jnp.concatenate([xi, xd], axis=1)
    m1, s1c, b1c = _mm(cat, p['msg'][0])
    y2, s2c, b2c = _mm(m1, p['msg'][1], affine=(s1c, b1c), slope=slope)
    l1, ls1, lb1 = _mm(x, p['lin'][0])
    l2, ls2, lb2 = _mm(l1, p['lin'][1], affine=(ls1, lb1), slope=slope)
    return _dyn_reduce(y2, s2c, b2c, l2, ls2, lb2, slope)


def kernel(node_embedding, edge_embedding, edge_index, k, params):
    src = edge_index[0]
    dst = edge_index[1]
    src32 = src.astype(jnp.int32)
    dst32 = dst.astype(jnp.int32)
    s1 = _static_conv_x(node_embedding, src, dst, edge_embedding,
                        params['sg1'], 0.1)
    s2 = _static_conv(s1, src32, dst32, edge_embedding, params['sg2'], 0.1)
    s3 = _static_conv(s2, src32, dst32, edge_embedding, params['sg3'], 0.1)
    nn1 = _knn(s1, 8)
    d1 = _dynamic_conv_x(s1, nn1, params['dg1'], 0.2)
    nn2 = _knn(d1, 8)
    d2 = _dynamic_conv(d1, nn2, params['dg2'], 0.2)
    cat1 = jnp.concatenate([s1, d1, d2, s2, s3], axis=1)
    f1r, f1s, f1b = _mm(cat1, params['fuse1'][0])
    f1 = _apply(f1r, f1s, f1b, 0.2)
    cat2 = jnp.concatenate([f1, cat1], axis=1)
    h1, h1s, h1b = _mm(cat2, params['fuse2'][0])
    h2, h2s, h2b = _mm(h1, params['fuse2'][1], affine=(h1s, h1b), slope=0.2)
    out, _, _ = _mm(h2, params['fuse2'][2], affine=(h2s, h2b), slope=0.2,
                    stats=False)
    return out
